# baseline (speedup 1.0000x reference)
import functools

import jax
import jax.numpy as jnp
from jax import lax
from jax.experimental import pallas as pl
from jax.experimental.pallas import tpu as pltpu
from jax.experimental.pallas import tpu_sc as plsc

_NC = 2
_NS = 16
_CHUNK = 100
_NBUF = 10
_RBLK = 1000


def _sc_edge_accumulate(hp, eidx):
    N, H = hp.shape
    n_rows = eidx.shape[1]
    nw = _NC * _NS
    rw = n_rows // nw
    n_grp = rw // _NBUF
    rps = N // _NS

    mesh = plsc.VectorSubcoreMesh(core_axis_name="c", subcore_axis_name="s")

    @functools.partial(
        pl.kernel,
        mesh=mesh,
        out_type=jax.ShapeDtypeStruct((2 * N, H), jnp.float32),
        compiler_params=pltpu.CompilerParams(use_tc_tiling_on_sc=False),
        scratch_types=[
            pltpu.VMEM((rw, _CHUNK), jnp.int32),
            pltpu.VMEM((rw, _CHUNK), jnp.int32),
            [pltpu.VMEM((_CHUNK, H), jnp.float32)] * _NBUF,
            pltpu.VMEM_SHARED((N, H), jnp.float32),
            pltpu.SemaphoreType.DMA,
        ],
    )
    def ker(hp_hbm, eidx_hbm, out_hbm,
            sidx, didx, rows, acc, sem):
        c = lax.axis_index("c")
        s = lax.axis_index("s")
        w = c * _NS + s
        zero16 = jnp.zeros((16,), jnp.float32)

        def zrow(i, carry):
            for j in range(H // 16):
                rows[0][i, pl.ds(j * 16, 16)] = zero16
            return carry

        lax.fori_loop(0, _CHUNK, zrow, 0)
        zfull, zrem = divmod(rps, _CHUNK)
        for z in range(zfull):
            pltpu.sync_copy(rows[0], acc.at[pl.ds(s * rps + z * _CHUNK, _CHUNK)])
        if zrem:
            pltpu.sync_copy(rows[0].at[pl.ds(0, zrem)],
                            acc.at[pl.ds(s * rps + zfull * _CHUNK, zrem)])
        pltpu.sync_copy(eidx_hbm.at[0, pl.ds(w * rw, rw)], sidx)
        pltpu.sync_copy(eidx_hbm.at[1, pl.ds(w * rw, rw)], didx)
        plsc.subcore_barrier()

        def group(g, carry):
            jb = g * _NBUF
            handles = [
                pltpu.async_copy(hp_hbm.at[sidx.at[jb + b]], rows[b], sem)
                for b in range(_NBUF)
            ]
            for b in range(_NBUF):
                handles[b].wait()
                pltpu.sync_copy(rows[b], acc.at[didx.at[jb + b]], add=True)
            return carry

        lax.fori_loop(0, n_grp, group, 0)
        plsc.subcore_barrier()

        pltpu.sync_copy(acc.at[pl.ds(s * rps, rps)],
                        out_hbm.at[pl.ds(c * N + s * rps, rps)])

    return ker(hp, eidx)


def _sc_edge_accumulate_fused(hp, dinv16, eidx):
    N, H = hp.shape
    W = dinv16.shape[1]
    n_rows = eidx.shape[1]
    nw = _NC * _NS
    rw = n_rows // nw
    nbuf = 5
    n_grp = rw // nbuf
    rps = N // _NS

    mesh = plsc.VectorSubcoreMesh(core_axis_name="c", subcore_axis_name="s")

    @functools.partial(
        pl.kernel,
        mesh=mesh,
        out_type=(jax.ShapeDtypeStruct((2 * N, H), jnp.float32),
                  jax.ShapeDtypeStruct((2 * N, W), jnp.float32)),
        compiler_params=pltpu.CompilerParams(use_tc_tiling_on_sc=False),
        scratch_types=[
            pltpu.VMEM((rw, _CHUNK), jnp.int32),
            pltpu.VMEM((rw, _CHUNK), jnp.int32),
            [pltpu.VMEM((_CHUNK, H), jnp.float32)] * nbuf,
            [pltpu.VMEM((_CHUNK, W), jnp.float32)] * nbuf,
            pltpu.VMEM_SHARED((N, H), jnp.float32),
            pltpu.VMEM_SHARED((N, W), jnp.float32),
            pltpu.SemaphoreType.DMA,
            pltpu.SemaphoreType.DMA,
        ],
    )
    def ker(hp_hbm, dinv_hbm, eidx_hbm, oute_hbm, outw_hbm,
            sidx, didx, rows, wrows, acce, accw, sem, semw):
        c = lax.axis_index("c")
        s = lax.axis_index("s")
        w = c * _NS + s
        zero16 = jnp.zeros((16,), jnp.float32)

        def zrow(i, carry):
            for j in range(H // 16):
                rows[0][i, pl.ds(j * 16, 16)] = zero16
            wrows[0][i, :] = zero16
            return carry

        lax.fori_loop(0, _CHUNK, zrow, 0)
        zfull, zrem = divmod(rps, _CHUNK)
        for z in range(zfull):
            pltpu.sync_copy(rows[0], acce.at[pl.ds(s * rps + z * _CHUNK, _CHUNK)])
            pltpu.sync_copy(wrows[0], accw.at[pl.ds(s * rps + z * _CHUNK, _CHUNK)])
        if zrem:
            pltpu.sync_copy(rows[0].at[pl.ds(0, zrem)],
                            acce.at[pl.ds(s * rps + zfull * _CHUNK, zrem)])
            pltpu.sync_copy(wrows[0].at[pl.ds(0, zrem)],
                            accw.at[pl.ds(s * rps + zfull * _CHUNK, zrem)])
        pltpu.sync_copy(eidx_hbm.at[0, pl.ds(w * rw, rw)], sidx)
        pltpu.sync_copy(eidx_hbm.at[1, pl.ds(w * rw, rw)], didx)
        plsc.subcore_barrier()

        def group(g, carry):
            jb = g * nbuf
            eh = [pltpu.async_copy(hp_hbm.at[sidx.at[jb + b]], rows[b], sem)
                  for b in range(nbuf)]
            wh = [pltpu.async_copy(dinv_hbm.at[didx.at[jb + b]], wrows[b], semw)
                  for b in range(nbuf)]
            for b in range(nbuf):
                eh[b].wait()
                pltpu.sync_copy(rows[b], acce.at[didx.at[jb + b]], add=True)
                wh[b].wait()
                pltpu.sync_copy(wrows[b], accw.at[sidx.at[jb + b]], add=True)
            return carry

        lax.fori_loop(0, n_grp, group, 0)
        plsc.subcore_barrier()

        pltpu.sync_copy(acce.at[pl.ds(s * rps, rps)],
                        oute_hbm.at[pl.ds(c * N + s * rps, rps)])
        pltpu.sync_copy(accw.at[pl.ds(s * rps, rps)],
                        outw_hbm.at[pl.ds(c * N + s * rps, rps)])

    return ker(hp, dinv16, eidx)


def _sc_degree(eidx, n_nodes):
    N = n_nodes
    W = 16
    n_rows = eidx.shape[1]
    nw = _NC * _NS
    rw = n_rows // nw
    rps = N // _NS

    mesh = plsc.VectorSubcoreMesh(core_axis_name="c", subcore_axis_name="s")

    @functools.partial(
        pl.kernel,
        mesh=mesh,
        out_type=jax.ShapeDtypeStruct((2 * N, W), jnp.float32),
        compiler_params=pltpu.CompilerParams(use_tc_tiling_on_sc=False),
        scratch_types=[
            pltpu.VMEM((rw, _CHUNK), jnp.int32),
            pltpu.VMEM((_CHUNK, W), jnp.float32),
            pltpu.VMEM((rps, W), jnp.float32),
            pltpu.VMEM_SHARED((N, W), jnp.float32),
            pltpu.SemaphoreType.DMA,
        ],
    )
    def ker(eidx_hbm, out_hbm, didx, ones, zbuf, acc, sem):
        c = lax.axis_index("c")
        s = lax.axis_index("s")
        w = c * _NS + s
        onehot = jnp.where(lax.iota(jnp.int32, 16) == 0,
                           jnp.float32(1), jnp.float32(0))
        zero16 = jnp.zeros((16,), jnp.float32)

        def fill(i, carry):
            ones[i, :] = onehot
            return carry

        lax.fori_loop(0, _CHUNK, fill, 0)

        def zrow(i, carry):
            zbuf[i, :] = zero16
            return carry

        lax.fori_loop(0, rps, zrow, 0)
        pltpu.sync_copy(zbuf, acc.at[pl.ds(s * rps, rps)])
        pltpu.sync_copy(eidx_hbm.at[1, pl.ds(w * rw, rw)], didx)
        plsc.subcore_barrier()

        def body(g, carry):
            jb = g * _NBUF
            handles = [
                pltpu.async_copy(ones, acc.at[didx.at[jb + b]], sem, add=True)
                for b in range(_NBUF)
            ]
            for h in handles:
                h.wait()
            return carry

        lax.fori_loop(0, rw // _NBUF, body, 0)
        plsc.subcore_barrier()

        pltpu.sync_copy(acc.at[pl.ds(s * rps, rps)],
                        out_hbm.at[pl.ds(c * N + s * rps, rps)])

    return ker(eidx)


def _tc_matmul(x, W1):
    N = x.shape[0]
    H = W1.shape[1]

    def body(x_ref, w_ref, out_ref):
        out_ref[...] = jnp.dot(x_ref[...], w_ref[...],
                               preferred_element_type=jnp.float32)

    nb = N // _RBLK
    return pl.pallas_call(
        body,
        grid=(nb,),
        in_specs=[pl.BlockSpec((_RBLK, x.shape[1]), lambda i: (i, 0)),
                  pl.BlockSpec(W1.shape, lambda i: (0, 0))],
        out_specs=pl.BlockSpec((_RBLK, H), lambda i: (i, 0)),
        out_shape=jax.ShapeDtypeStruct((N, H), jnp.float32),
    )(x, W1)


def _tc_first(degp, xw):
    _, N, _ = degp.shape
    H = xw.shape[1]

    def body(deg_ref, xw_ref, hp_ref, dinv_ref, dinv16_ref):
        deg = deg_ref[0, :, 0:1] + deg_ref[1, :, 0:1] + 1.0
        dinv = lax.rsqrt(deg)
        dinv_ref[...] = dinv
        dinv16_ref[...] = jnp.broadcast_to(dinv, (_RBLK, 16))
        hp_ref[...] = xw_ref[...] * dinv

    nb = N // _RBLK
    return pl.pallas_call(
        body,
        grid=(nb,),
        in_specs=[pl.BlockSpec((2, _RBLK, 16), lambda i: (0, i, 0)),
                  pl.BlockSpec((_RBLK, H), lambda i: (i, 0))],
        out_specs=(pl.BlockSpec((_RBLK, H), lambda i: (i, 0)),
                   pl.BlockSpec((_RBLK, 1), lambda i: (i, 0)),
                   pl.BlockSpec((_RBLK, 16), lambda i: (i, 0))),
        out_shape=(jax.ShapeDtypeStruct((N, H), jnp.float32),
                   jax.ShapeDtypeStruct((N, 1), jnp.float32),
                   jax.ShapeDtypeStruct((N, 16), jnp.float32)),
    )(degp, xw)


def _tc_mid(e, hp, dinv, b, W):
    _, N, H = e.shape
    HO = W.shape[1]

    def body(e_ref, hp_ref, dinv_ref, b_ref, w_ref, out_ref):
        esum = e_ref[0] + e_ref[1] + hp_ref[...]
        h = jnp.maximum(esum * dinv_ref[...] + b_ref[...], 0.0)
        out_ref[...] = jnp.dot(h, w_ref[...],
                               preferred_element_type=jnp.float32) * dinv_ref[...]

    nb = N // _RBLK
    return pl.pallas_call(
        body,
        grid=(nb,),
        in_specs=[pl.BlockSpec((2, _RBLK, H), lambda i: (0, i, 0)),
                  pl.BlockSpec((_RBLK, H), lambda i: (i, 0)),
                  pl.BlockSpec((_RBLK, 1), lambda i: (i, 0)),
                  pl.BlockSpec((1, H), lambda i: (0, 0)),
                  pl.BlockSpec(W.shape, lambda i: (0, 0))],
        out_specs=pl.BlockSpec((_RBLK, HO), lambda i: (i, 0)),
        out_shape=jax.ShapeDtypeStruct((N, HO), jnp.float32),
    )(e, hp, dinv, b, W)


def _tc_final(e, hp, dinv, b2, W3, wsump, b3, Wc1, bc1, Wc2, bc2):
    _, N, H = e.shape
    nb = N // _RBLK

    def body(e_ref, hp_ref, dinv_ref, b2_ref, w3_ref, w_ref, b3_ref,
             wc1_ref, bc1_ref, wc2_ref, bc2_ref, out_ref, u_acc):
        i = pl.program_id(0)
        dinv = dinv_ref[...]
        esum = e_ref[0] + e_ref[1] + hp_ref[...]
        h2 = jnp.maximum(esum * dinv + b2_ref[...], 0.0)
        v = (w_ref[0, :, 0:1] + w_ref[1, :, 0:1] + dinv) * dinv
        u = jnp.sum(h2 * v, axis=0, keepdims=True)

        @pl.when(i == 0)
        def _():
            u_acc[...] = jnp.zeros_like(u_acc)

        u_acc[...] += u

        @pl.when(i == nb - 1)
        def _():
            g = jnp.dot(u_acc[...] * jnp.float32(1.0 / N), w3_ref[...],
                        preferred_element_type=jnp.float32) + b3_ref[...]
            z = jnp.maximum(jnp.dot(g, wc1_ref[...],
                                    preferred_element_type=jnp.float32)
                            + bc1_ref[...], 0.0)
            out_ref[...] = jnp.dot(z, wc2_ref[...],
                                   preferred_element_type=jnp.float32) + bc2_ref[...]

    return pl.pallas_call(
        body,
        grid=(nb,),
        in_specs=[pl.BlockSpec((2, _RBLK, H), lambda i: (0, i, 0)),
                  pl.BlockSpec((_RBLK, H), lambda i: (i, 0)),
                  pl.BlockSpec((_RBLK, 1), lambda i: (i, 0)),
                  pl.BlockSpec((1, H), lambda i: (0, 0)),
                  pl.BlockSpec(W3.shape, lambda i: (0, 0)),
                  pl.BlockSpec((2, _RBLK, 16), lambda i: (0, i, 0)),
                  pl.BlockSpec((1, H), lambda i: (0, 0)),
                  pl.BlockSpec(Wc1.shape, lambda i: (0, 0)),
                  pl.BlockSpec((1, Wc1.shape[1]), lambda i: (0, 0)),
                  pl.BlockSpec(Wc2.shape, lambda i: (0, 0)),
                  pl.BlockSpec((1, Wc2.shape[1]), lambda i: (0, 0))],
        out_specs=pl.BlockSpec((1, Wc2.shape[1]), lambda i: (0, 0)),
        out_shape=jax.ShapeDtypeStruct((1, Wc2.shape[1]), jnp.float32),
        scratch_shapes=[pltpu.VMEM((1, H), jnp.float32)],
    )(e, hp, dinv, b2, W3, wsump, b3, Wc1, bc1, Wc2, bc2)


def kernel(x, edge_index, W1, b1, W2, b2, W3, b3, Wc1, bc1, Wc2, bc2):
    N = x.shape[0]
    H = W1.shape[1]
    eidx = edge_index.reshape(2, -1, _CHUNK)

    xw1 = _tc_matmul(x, W1)
    degp = _sc_degree(eidx, N)
    h1p, dinv, dinv16 = _tc_first(degp.reshape(2, N, 16), xw1)

    e1, wsump = _sc_edge_accumulate_fused(h1p, dinv16, eidx)
    h2p = _tc_mid(e1.reshape(2, N, H), h1p, dinv, b1.reshape(1, -1), W2)

    e2 = _sc_edge_accumulate(h2p, eidx)
    out = _tc_final(e2.reshape(2, N, H), h2p, dinv, b2.reshape(1, -1), W3,
                    wsump.reshape(2, N, 16), b3.reshape(1, -1), Wc1,
                    bc1.reshape(1, -1), Wc2, bc2.reshape(1, -1))
    return out


# speedup vs baseline: 1.0493x; 1.0493x over previous
import functools

import jax
import jax.numpy as jnp
from jax import lax
from jax.experimental import pallas as pl
from jax.experimental.pallas import tpu as pltpu
from jax.experimental.pallas import tpu_sc as plsc

_NC = 2
_NS = 16
_CHUNK = 128
_RBLK = 1000


def _sc_edge_accumulate(hp, eidx):
    N, H = hp.shape
    E = eidx.shape[1]
    nw = _NC * _NS
    per_w = E // nw
    rw, tail = divmod(per_w, _CHUNK)
    nbuf = 6
    n_grp = rw // nbuf
    n_rem = rw - n_grp * nbuf
    rps = N // _NS

    mesh = plsc.VectorSubcoreMesh(core_axis_name="c", subcore_axis_name="s")

    @functools.partial(
        pl.kernel,
        mesh=mesh,
        out_type=jax.ShapeDtypeStruct((2 * N, H), jnp.float32),
        compiler_params=pltpu.CompilerParams(use_tc_tiling_on_sc=False),
        scratch_types=[
            pltpu.VMEM((per_w,), jnp.int32),
            pltpu.VMEM((per_w,), jnp.int32),
            [pltpu.VMEM((_CHUNK, H), jnp.float32)] * nbuf,
            pltpu.VMEM_SHARED((N, H), jnp.float32),
            pltpu.SemaphoreType.DMA,
        ],
    )
    def ker(hp_hbm, eidx_hbm, out_hbm,
            sidx, didx, rows, acc, sem):
        c = lax.axis_index("c")
        s = lax.axis_index("s")
        w = c * _NS + s
        zero16 = jnp.zeros((16,), jnp.float32)

        def zrow(i, carry):
            for j in range(H // 16):
                rows[0][i, pl.ds(j * 16, 16)] = zero16
            return carry

        lax.fori_loop(0, _CHUNK, zrow, 0)
        zfull, zrem = divmod(rps, _CHUNK)
        for z in range(zfull):
            pltpu.sync_copy(rows[0], acc.at[pl.ds(s * rps + z * _CHUNK, _CHUNK)])
        if zrem:
            pltpu.sync_copy(rows[0].at[pl.ds(0, zrem)],
                            acc.at[pl.ds(s * rps + zfull * _CHUNK, zrem)])
        pltpu.sync_copy(eidx_hbm.at[0, pl.ds(w * per_w, per_w)], sidx)
        pltpu.sync_copy(eidx_hbm.at[1, pl.ds(w * per_w, per_w)], didx)
        plsc.subcore_barrier()

        def run_chunks(jb, nb, size):
            handles = [
                pltpu.async_copy(
                    hp_hbm.at[sidx.at[pl.ds((jb + b) * _CHUNK, size)]],
                    rows[b].at[pl.ds(0, size)], sem)
                for b in range(nb)
            ]
            for b in range(nb):
                handles[b].wait()
                pltpu.sync_copy(
                    rows[b].at[pl.ds(0, size)],
                    acc.at[didx.at[pl.ds((jb + b) * _CHUNK, size)]],
                    add=True)

        def group(g, carry):
            run_chunks(g * nbuf, nbuf, _CHUNK)
            return carry

        lax.fori_loop(0, n_grp, group, 0)
        if n_rem:
            run_chunks(n_grp * nbuf, n_rem, _CHUNK)
        if tail:
            run_chunks(rw, 1, tail)
        plsc.subcore_barrier()

        pltpu.sync_copy(acc.at[pl.ds(s * rps, rps)],
                        out_hbm.at[pl.ds(c * N + s * rps, rps)])

    return ker(hp, eidx)


def _sc_edge_accumulate_fused(hp, dinv16, eidx):
    N, H = hp.shape
    W = dinv16.shape[1]
    E = eidx.shape[1]
    nw = _NC * _NS
    per_w = E // nw
    rw, tail = divmod(per_w, _CHUNK)
    nbuf = 5
    n_grp = rw // nbuf
    n_rem = rw - n_grp * nbuf
    rps = N // _NS

    mesh = plsc.VectorSubcoreMesh(core_axis_name="c", subcore_axis_name="s")

    @functools.partial(
        pl.kernel,
        mesh=mesh,
        out_type=(jax.ShapeDtypeStruct((2 * N, H), jnp.float32),
                  jax.ShapeDtypeStruct((2 * N, W), jnp.float32)),
        compiler_params=pltpu.CompilerParams(use_tc_tiling_on_sc=False),
        scratch_types=[
            pltpu.VMEM((per_w,), jnp.int32),
            pltpu.VMEM((per_w,), jnp.int32),
            [pltpu.VMEM((_CHUNK, H), jnp.float32)] * nbuf,
            [pltpu.VMEM((_CHUNK, W), jnp.float32)] * nbuf,
            pltpu.VMEM_SHARED((N, H), jnp.float32),
            pltpu.VMEM_SHARED((N, W), jnp.float32),
            pltpu.SemaphoreType.DMA,
            pltpu.SemaphoreType.DMA,
        ],
    )
    def ker(hp_hbm, dinv_hbm, eidx_hbm, oute_hbm, outw_hbm,
            sidx, didx, rows, wrows, acce, accw, sem, semw):
        c = lax.axis_index("c")
        s = lax.axis_index("s")
        w = c * _NS + s
        zero16 = jnp.zeros((16,), jnp.float32)

        def zrow(i, carry):
            for j in range(H // 16):
                rows[0][i, pl.ds(j * 16, 16)] = zero16
            wrows[0][i, :] = zero16
            return carry

        lax.fori_loop(0, _CHUNK, zrow, 0)
        zfull, zrem = divmod(rps, _CHUNK)
        for z in range(zfull):
            pltpu.sync_copy(rows[0], acce.at[pl.ds(s * rps + z * _CHUNK, _CHUNK)])
            pltpu.sync_copy(wrows[0], accw.at[pl.ds(s * rps + z * _CHUNK, _CHUNK)])
        if zrem:
            pltpu.sync_copy(rows[0].at[pl.ds(0, zrem)],
                            acce.at[pl.ds(s * rps + zfull * _CHUNK, zrem)])
            pltpu.sync_copy(wrows[0].at[pl.ds(0, zrem)],
                            accw.at[pl.ds(s * rps + zfull * _CHUNK, zrem)])
        pltpu.sync_copy(eidx_hbm.at[0, pl.ds(w * per_w, per_w)], sidx)
        pltpu.sync_copy(eidx_hbm.at[1, pl.ds(w * per_w, per_w)], didx)
        plsc.subcore_barrier()

        def run_chunks(jb, nb, size):
            def sl(idx_ref, b):
                return idx_ref.at[pl.ds((jb + b) * _CHUNK, size)]

            eh = [pltpu.async_copy(hp_hbm.at[sl(sidx, b)],
                                   rows[b].at[pl.ds(0, size)], sem)
                  for b in range(nb)]
            wh = [pltpu.async_copy(dinv_hbm.at[sl(didx, b)],
                                   wrows[b].at[pl.ds(0, size)], semw)
                  for b in range(nb)]
            for b in range(nb):
                eh[b].wait()
                pltpu.sync_copy(rows[b].at[pl.ds(0, size)],
                                acce.at[sl(didx, b)], add=True)
                wh[b].wait()
                pltpu.sync_copy(wrows[b].at[pl.ds(0, size)],
                                accw.at[sl(sidx, b)], add=True)

        def group(g, carry):
            run_chunks(g * nbuf, nbuf, _CHUNK)
            return carry

        lax.fori_loop(0, n_grp, group, 0)
        if n_rem:
            run_chunks(n_grp * nbuf, n_rem, _CHUNK)
        if tail:
            run_chunks(rw, 1, tail)
        plsc.subcore_barrier()

        pltpu.sync_copy(acce.at[pl.ds(s * rps, rps)],
                        oute_hbm.at[pl.ds(c * N + s * rps, rps)])
        pltpu.sync_copy(accw.at[pl.ds(s * rps, rps)],
                        outw_hbm.at[pl.ds(c * N + s * rps, rps)])

    return ker(hp, dinv16, eidx)


def _sc_degree(eidx, n_nodes):
    N = n_nodes
    W = 16
    E = eidx.shape[1]
    nw = _NC * _NS
    per_w = E // nw
    rw, tail = divmod(per_w, _CHUNK)
    nbuf = 6
    n_grp = rw // nbuf
    n_rem = rw - n_grp * nbuf
    rps = N // _NS

    mesh = plsc.VectorSubcoreMesh(core_axis_name="c", subcore_axis_name="s")

    @functools.partial(
        pl.kernel,
        mesh=mesh,
        out_type=jax.ShapeDtypeStruct((2 * N, W), jnp.float32),
        compiler_params=pltpu.CompilerParams(use_tc_tiling_on_sc=False),
        scratch_types=[
            pltpu.VMEM((per_w,), jnp.int32),
            pltpu.VMEM((_CHUNK, W), jnp.float32),
            pltpu.VMEM((rps, W), jnp.float32),
            pltpu.VMEM_SHARED((N, W), jnp.float32),
            pltpu.SemaphoreType.DMA,
        ],
    )
    def ker(eidx_hbm, out_hbm, didx, ones, zbuf, acc, sem):
        c = lax.axis_index("c")
        s = lax.axis_index("s")
        w = c * _NS + s
        onehot = jnp.where(lax.iota(jnp.int32, 16) == 0,
                           jnp.float32(1), jnp.float32(0))
        zero16 = jnp.zeros((16,), jnp.float32)

        def fill(i, carry):
            ones[i, :] = onehot
            return carry

        lax.fori_loop(0, _CHUNK, fill, 0)

        def zrow(i, carry):
            zbuf[i, :] = zero16
            return carry

        lax.fori_loop(0, rps, zrow, 0)
        pltpu.sync_copy(zbuf, acc.at[pl.ds(s * rps, rps)])
        pltpu.sync_copy(eidx_hbm.at[1, pl.ds(w * per_w, per_w)], didx)
        plsc.subcore_barrier()

        def run_chunks(jb, nb, size):
            handles = [
                pltpu.async_copy(
                    ones.at[pl.ds(0, size)],
                    acc.at[didx.at[pl.ds((jb + b) * _CHUNK, size)]],
                    sem, add=True)
                for b in range(nb)
            ]
            for h in handles:
                h.wait()

        def body(g, carry):
            run_chunks(g * nbuf, nbuf, _CHUNK)
            return carry

        lax.fori_loop(0, n_grp, body, 0)
        if n_rem:
            run_chunks(n_grp * nbuf, n_rem, _CHUNK)
        if tail:
            run_chunks(rw, 1, tail)
        plsc.subcore_barrier()

        pltpu.sync_copy(acc.at[pl.ds(s * rps, rps)],
                        out_hbm.at[pl.ds(c * N + s * rps, rps)])

    return ker(eidx)


def _tc_matmul(x, W1):
    N = x.shape[0]
    H = W1.shape[1]

    def body(x_ref, w_ref, out_ref):
        out_ref[...] = jnp.dot(x_ref[...], w_ref[...],
                               preferred_element_type=jnp.float32)

    nb = N // _RBLK
    return pl.pallas_call(
        body,
        grid=(nb,),
        in_specs=[pl.BlockSpec((_RBLK, x.shape[1]), lambda i: (i, 0)),
                  pl.BlockSpec(W1.shape, lambda i: (0, 0))],
        out_specs=pl.BlockSpec((_RBLK, H), lambda i: (i, 0)),
        out_shape=jax.ShapeDtypeStruct((N, H), jnp.float32),
    )(x, W1)


def _tc_first(degp, xw):
    _, N, _ = degp.shape
    H = xw.shape[1]

    def body(deg_ref, xw_ref, hp_ref, dinv_ref, dinv16_ref):
        deg = deg_ref[0, :, 0:1] + deg_ref[1, :, 0:1] + 1.0
        dinv = lax.rsqrt(deg)
        dinv_ref[...] = dinv
        dinv16_ref[...] = jnp.broadcast_to(dinv, (_RBLK, 16))
        hp_ref[...] = xw_ref[...] * dinv

    nb = N // _RBLK
    return pl.pallas_call(
        body,
        grid=(nb,),
        in_specs=[pl.BlockSpec((2, _RBLK, 16), lambda i: (0, i, 0)),
                  pl.BlockSpec((_RBLK, H), lambda i: (i, 0))],
        out_specs=(pl.BlockSpec((_RBLK, H), lambda i: (i, 0)),
                   pl.BlockSpec((_RBLK, 1), lambda i: (i, 0)),
                   pl.BlockSpec((_RBLK, 16), lambda i: (i, 0))),
        out_shape=(jax.ShapeDtypeStruct((N, H), jnp.float32),
                   jax.ShapeDtypeStruct((N, 1), jnp.float32),
                   jax.ShapeDtypeStruct((N, 16), jnp.float32)),
    )(degp, xw)


def _tc_mid(e, hp, dinv, b, W):
    _, N, H = e.shape
    HO = W.shape[1]

    def body(e_ref, hp_ref, dinv_ref, b_ref, w_ref, out_ref):
        esum = e_ref[0] + e_ref[1] + hp_ref[...]
        h = jnp.maximum(esum * dinv_ref[...] + b_ref[...], 0.0)
        out_ref[...] = jnp.dot(h, w_ref[...],
                               preferred_element_type=jnp.float32) * dinv_ref[...]

    nb = N // _RBLK
    return pl.pallas_call(
        body,
        grid=(nb,),
        in_specs=[pl.BlockSpec((2, _RBLK, H), lambda i: (0, i, 0)),
                  pl.BlockSpec((_RBLK, H), lambda i: (i, 0)),
                  pl.BlockSpec((_RBLK, 1), lambda i: (i, 0)),
                  pl.BlockSpec((1, H), lambda i: (0, 0)),
                  pl.BlockSpec(W.shape, lambda i: (0, 0))],
        out_specs=pl.BlockSpec((_RBLK, HO), lambda i: (i, 0)),
        out_shape=jax.ShapeDtypeStruct((N, HO), jnp.float32),
    )(e, hp, dinv, b, W)


def _tc_final(e, hp, dinv, b2, W3, wsump, b3, Wc1, bc1, Wc2, bc2):
    _, N, H = e.shape
    nb = N // _RBLK

    def body(e_ref, hp_ref, dinv_ref, b2_ref, w3_ref, w_ref, b3_ref,
             wc1_ref, bc1_ref, wc2_ref, bc2_ref, out_ref, u_acc):
        i = pl.program_id(0)
        dinv = dinv_ref[...]
        esum = e_ref[0] + e_ref[1] + hp_ref[...]
        h2 = jnp.maximum(esum * dinv + b2_ref[...], 0.0)
        v = (w_ref[0, :, 0:1] + w_ref[1, :, 0:1] + dinv) * dinv
        u = jnp.sum(h2 * v, axis=0, keepdims=True)

        @pl.when(i == 0)
        def _():
            u_acc[...] = jnp.zeros_like(u_acc)

        u_acc[...] += u

        @pl.when(i == nb - 1)
        def _():
            g = jnp.dot(u_acc[...] * jnp.float32(1.0 / N), w3_ref[...],
                        preferred_element_type=jnp.float32) + b3_ref[...]
            z = jnp.maximum(jnp.dot(g, wc1_ref[...],
                                    preferred_element_type=jnp.float32)
                            + bc1_ref[...], 0.0)
            out_ref[...] = jnp.dot(z, wc2_ref[...],
                                   preferred_element_type=jnp.float32) + bc2_ref[...]

    return pl.pallas_call(
        body,
        grid=(nb,),
        in_specs=[pl.BlockSpec((2, _RBLK, H), lambda i: (0, i, 0)),
                  pl.BlockSpec((_RBLK, H), lambda i: (i, 0)),
                  pl.BlockSpec((_RBLK, 1), lambda i: (i, 0)),
                  pl.BlockSpec((1, H), lambda i: (0, 0)),
                  pl.BlockSpec(W3.shape, lambda i: (0, 0)),
                  pl.BlockSpec((2, _RBLK, 16), lambda i: (0, i, 0)),
                  pl.BlockSpec((1, H), lambda i: (0, 0)),
                  pl.BlockSpec(Wc1.shape, lambda i: (0, 0)),
                  pl.BlockSpec((1, Wc1.shape[1]), lambda i: (0, 0)),
                  pl.BlockSpec(Wc2.shape, lambda i: (0, 0)),
                  pl.BlockSpec((1, Wc2.shape[1]), lambda i: (0, 0))],
        out_specs=pl.BlockSpec((1, Wc2.shape[1]), lambda i: (0, 0)),
        out_shape=jax.ShapeDtypeStruct((1, Wc2.shape[1]), jnp.float32),
        scratch_shapes=[pltpu.VMEM((1, H), jnp.float32)],
    )(e, hp, dinv, b2, W3, wsump, b3, Wc1, bc1, Wc2, bc2)


def kernel(x, edge_index, W1, b1, W2, b2, W3, b3, Wc1, bc1, Wc2, bc2):
    N = x.shape[0]
    H = W1.shape[1]
    eidx = edge_index

    xw1 = _tc_matmul(x, W1)
    degp = _sc_degree(eidx, N)
    h1p, dinv, dinv16 = _tc_first(degp.reshape(2, N, 16), xw1)

    e1, wsump = _sc_edge_accumulate_fused(h1p, dinv16, eidx)
    h2p = _tc_mid(e1.reshape(2, N, H), h1p, dinv, b1.reshape(1, -1), W2)

    e2 = _sc_edge_accumulate(h2p, eidx)
    out = _tc_final(e2.reshape(2, N, H), h2p, dinv, b2.reshape(1, -1), W3,
                    wsump.reshape(2, N, 16), b3.reshape(1, -1), Wc1,
                    bc1.reshape(1, -1), Wc2, bc2.reshape(1, -1))
    return out


# speedup vs baseline: 1.1250x; 1.0721x over previous
import functools

import jax
import jax.numpy as jnp
from jax import lax
from jax.experimental import pallas as pl
from jax.experimental.pallas import tpu as pltpu
from jax.experimental.pallas import tpu_sc as plsc

_NC = 2
_NS = 16
_CHUNK = 128
_RBLK = 1000


def _sc_edge_accumulate(hp, eidx):
    N, H = hp.shape
    E = eidx.shape[1]
    nw = _NC * _NS
    per_w = E // nw
    rw, tail = divmod(per_w, _CHUNK)
    nbuf = 6
    n_grp = rw // nbuf
    n_rem = rw - n_grp * nbuf
    rps = N // _NS

    mesh = plsc.VectorSubcoreMesh(core_axis_name="c", subcore_axis_name="s")

    @functools.partial(
        pl.kernel,
        mesh=mesh,
        out_type=jax.ShapeDtypeStruct((2 * N, 128), jnp.float32),
        compiler_params=pltpu.CompilerParams(use_tc_tiling_on_sc=False),
        scratch_types=[
            pltpu.VMEM((per_w,), jnp.int32),
            pltpu.VMEM((per_w,), jnp.int32),
            [pltpu.VMEM((_CHUNK, H), jnp.float32)] * nbuf,
            pltpu.VMEM_SHARED((N, H), jnp.float32),
            pltpu.SemaphoreType.DMA,
        ],
    )
    def ker(hp_hbm, eidx_hbm, out_hbm,
            sidx, didx, rows, acc, sem):
        c = lax.axis_index("c")
        s = lax.axis_index("s")
        w = c * _NS + s
        zero16 = jnp.zeros((16,), jnp.float32)

        def zrow(i, carry):
            for j in range(H // 16):
                rows[0][i, pl.ds(j * 16, 16)] = zero16
            return carry

        lax.fori_loop(0, _CHUNK, zrow, 0)
        zfull, zrem = divmod(rps, _CHUNK)
        for z in range(zfull):
            pltpu.sync_copy(rows[0], acc.at[pl.ds(s * rps + z * _CHUNK, _CHUNK)])
        if zrem:
            pltpu.sync_copy(rows[0].at[pl.ds(0, zrem)],
                            acc.at[pl.ds(s * rps + zfull * _CHUNK, zrem)])
        pltpu.sync_copy(eidx_hbm.at[0, pl.ds(w * per_w, per_w)], sidx)
        pltpu.sync_copy(eidx_hbm.at[1, pl.ds(w * per_w, per_w)], didx)
        plsc.subcore_barrier()

        def run_chunks(jb, nb, size):
            handles = [
                pltpu.async_copy(
                    hp_hbm.at[sidx.at[pl.ds((jb + b) * _CHUNK, size)]],
                    rows[b].at[pl.ds(0, size)], sem)
                for b in range(nb)
            ]
            for b in range(nb):
                handles[b].wait()
                pltpu.sync_copy(
                    rows[b].at[pl.ds(0, size)],
                    acc.at[didx.at[pl.ds((jb + b) * _CHUNK, size)]],
                    add=True)

        def group(g, carry):
            run_chunks(g * nbuf, nbuf, _CHUNK)
            return carry

        lax.fori_loop(0, n_grp, group, 0)
        if n_rem:
            run_chunks(n_grp * nbuf, n_rem, _CHUNK)
        if tail:
            run_chunks(rw, 1, tail)
        plsc.subcore_barrier()

        pltpu.sync_copy(acc.at[pl.ds(s * rps, rps)],
                        out_hbm.at[pl.ds(c * N + s * rps, rps), pl.ds(0, H)])

    return ker(hp, eidx)


def _sc_edge_accumulate_fused(hp, dinv16, eidx):
    N, H = hp.shape
    W = dinv16.shape[1]
    E = eidx.shape[1]
    nw = _NC * _NS
    per_w = E // nw
    rw, tail = divmod(per_w, _CHUNK)
    nbuf = 5
    n_grp = rw // nbuf
    n_rem = rw - n_grp * nbuf
    rps = N // _NS

    mesh = plsc.VectorSubcoreMesh(core_axis_name="c", subcore_axis_name="s")

    @functools.partial(
        pl.kernel,
        mesh=mesh,
        out_type=(jax.ShapeDtypeStruct((2 * N, 128), jnp.float32),
                  jax.ShapeDtypeStruct((2 * N, W), jnp.float32)),
        compiler_params=pltpu.CompilerParams(use_tc_tiling_on_sc=False),
        scratch_types=[
            pltpu.VMEM((per_w,), jnp.int32),
            pltpu.VMEM((per_w,), jnp.int32),
            [pltpu.VMEM((_CHUNK, H), jnp.float32)] * nbuf,
            [pltpu.VMEM((_CHUNK, W), jnp.float32)] * nbuf,
            pltpu.VMEM_SHARED((N, H), jnp.float32),
            pltpu.VMEM_SHARED((N, W), jnp.float32),
            pltpu.SemaphoreType.DMA,
            pltpu.SemaphoreType.DMA,
        ],
    )
    def ker(hp_hbm, dinv_hbm, eidx_hbm, oute_hbm, outw_hbm,
            sidx, didx, rows, wrows, acce, accw, sem, semw):
        c = lax.axis_index("c")
        s = lax.axis_index("s")
        w = c * _NS + s
        zero16 = jnp.zeros((16,), jnp.float32)

        def zrow(i, carry):
            for j in range(H // 16):
                rows[0][i, pl.ds(j * 16, 16)] = zero16
            wrows[0][i, :] = zero16
            return carry

        lax.fori_loop(0, _CHUNK, zrow, 0)
        zfull, zrem = divmod(rps, _CHUNK)
        for z in range(zfull):
            pltpu.sync_copy(rows[0], acce.at[pl.ds(s * rps + z * _CHUNK, _CHUNK)])
            pltpu.sync_copy(wrows[0], accw.at[pl.ds(s * rps + z * _CHUNK, _CHUNK)])
        if zrem:
            pltpu.sync_copy(rows[0].at[pl.ds(0, zrem)],
                            acce.at[pl.ds(s * rps + zfull * _CHUNK, zrem)])
            pltpu.sync_copy(wrows[0].at[pl.ds(0, zrem)],
                            accw.at[pl.ds(s * rps + zfull * _CHUNK, zrem)])
        pltpu.sync_copy(eidx_hbm.at[0, pl.ds(w * per_w, per_w)], sidx)
        pltpu.sync_copy(eidx_hbm.at[1, pl.ds(w * per_w, per_w)], didx)
        plsc.subcore_barrier()

        def run_chunks(jb, nb, size):
            def sl(idx_ref, b):
                return idx_ref.at[pl.ds((jb + b) * _CHUNK, size)]

            eh = [pltpu.async_copy(hp_hbm.at[sl(sidx, b)],
                                   rows[b].at[pl.ds(0, size)], sem)
                  for b in range(nb)]
            wh = [pltpu.async_copy(dinv_hbm.at[sl(didx, b)],
                                   wrows[b].at[pl.ds(0, size)], semw)
                  for b in range(nb)]
            for b in range(nb):
                eh[b].wait()
                pltpu.sync_copy(rows[b].at[pl.ds(0, size)],
                                acce.at[sl(didx, b)], add=True)
                wh[b].wait()
                pltpu.sync_copy(wrows[b].at[pl.ds(0, size)],
                                accw.at[sl(sidx, b)], add=True)

        def group(g, carry):
            run_chunks(g * nbuf, nbuf, _CHUNK)
            return carry

        lax.fori_loop(0, n_grp, group, 0)
        if n_rem:
            run_chunks(n_grp * nbuf, n_rem, _CHUNK)
        if tail:
            run_chunks(rw, 1, tail)
        plsc.subcore_barrier()

        pltpu.sync_copy(acce.at[pl.ds(s * rps, rps)],
                        oute_hbm.at[pl.ds(c * N + s * rps, rps), pl.ds(0, H)])
        pltpu.sync_copy(accw.at[pl.ds(s * rps, rps)],
                        outw_hbm.at[pl.ds(c * N + s * rps, rps)])

    return ker(hp, dinv16, eidx)


def _sc_degree(eidx, n_nodes):
    N = n_nodes
    W = 16
    E = eidx.shape[1]
    nw = _NC * _NS
    per_w = E // nw
    rw, tail = divmod(per_w, _CHUNK)
    nbuf = 6
    n_grp = rw // nbuf
    n_rem = rw - n_grp * nbuf
    rps = N // _NS

    mesh = plsc.VectorSubcoreMesh(core_axis_name="c", subcore_axis_name="s")

    @functools.partial(
        pl.kernel,
        mesh=mesh,
        out_type=jax.ShapeDtypeStruct((2 * N, W), jnp.float32),
        compiler_params=pltpu.CompilerParams(use_tc_tiling_on_sc=False),
        scratch_types=[
            pltpu.VMEM((per_w,), jnp.int32),
            pltpu.VMEM((_CHUNK, W), jnp.float32),
            pltpu.VMEM((rps, W), jnp.float32),
            pltpu.VMEM_SHARED((N, W), jnp.float32),
            pltpu.SemaphoreType.DMA,
        ],
    )
    def ker(eidx_hbm, out_hbm, didx, ones, zbuf, acc, sem):
        c = lax.axis_index("c")
        s = lax.axis_index("s")
        w = c * _NS + s
        onehot = jnp.where(lax.iota(jnp.int32, 16) == 0,
                           jnp.float32(1), jnp.float32(0))
        zero16 = jnp.zeros((16,), jnp.float32)

        def fill(i, carry):
            ones[i, :] = onehot
            return carry

        lax.fori_loop(0, _CHUNK, fill, 0)

        def zrow(i, carry):
            zbuf[i, :] = zero16
            return carry

        lax.fori_loop(0, rps, zrow, 0)
        pltpu.sync_copy(zbuf, acc.at[pl.ds(s * rps, rps)])
        pltpu.sync_copy(eidx_hbm.at[1, pl.ds(w * per_w, per_w)], didx)
        plsc.subcore_barrier()

        def run_chunks(jb, nb, size):
            handles = [
                pltpu.async_copy(
                    ones.at[pl.ds(0, size)],
                    acc.at[didx.at[pl.ds((jb + b) * _CHUNK, size)]],
                    sem, add=True)
                for b in range(nb)
            ]
            for h in handles:
                h.wait()

        def body(g, carry):
            run_chunks(g * nbuf, nbuf, _CHUNK)
            return carry

        lax.fori_loop(0, n_grp, body, 0)
        if n_rem:
            run_chunks(n_grp * nbuf, n_rem, _CHUNK)
        if tail:
            run_chunks(rw, 1, tail)
        plsc.subcore_barrier()

        pltpu.sync_copy(acc.at[pl.ds(s * rps, rps)],
                        out_hbm.at[pl.ds(c * N + s * rps, rps)])

    return ker(eidx)


def _tc_matmul(x, W1):
    N = x.shape[0]
    H = W1.shape[1]

    def body(x_ref, w_ref, out_ref):
        out_ref[...] = jnp.dot(x_ref[...], w_ref[...],
                               preferred_element_type=jnp.float32)

    nb = N // _RBLK
    return pl.pallas_call(
        body,
        grid=(nb,),
        in_specs=[pl.BlockSpec((_RBLK, x.shape[1]), lambda i: (i, 0)),
                  pl.BlockSpec(W1.shape, lambda i: (0, 0))],
        out_specs=pl.BlockSpec((_RBLK, H), lambda i: (i, 0)),
        out_shape=jax.ShapeDtypeStruct((N, H), jnp.float32),
    )(x, W1)


def _tc_first(degp, xw):
    _, N, _ = degp.shape
    H = xw.shape[1]

    def body(deg_ref, xw_ref, hp_ref, dinv_ref, dinv16_ref):
        deg = deg_ref[0, :, 0:1] + deg_ref[1, :, 0:1] + 1.0
        dinv = lax.rsqrt(deg)
        dinv_ref[...] = dinv
        dinv16_ref[...] = jnp.broadcast_to(dinv, (_RBLK, 16))
        hp_ref[...] = xw_ref[...] * dinv

    nb = N // _RBLK
    return pl.pallas_call(
        body,
        grid=(nb,),
        in_specs=[pl.BlockSpec((2, _RBLK, 16), lambda i: (0, i, 0)),
                  pl.BlockSpec((_RBLK, H), lambda i: (i, 0))],
        out_specs=(pl.BlockSpec((_RBLK, H), lambda i: (i, 0)),
                   pl.BlockSpec((_RBLK, 1), lambda i: (i, 0)),
                   pl.BlockSpec((_RBLK, 16), lambda i: (i, 0))),
        out_shape=(jax.ShapeDtypeStruct((N, H), jnp.float32),
                   jax.ShapeDtypeStruct((N, 1), jnp.float32),
                   jax.ShapeDtypeStruct((N, 16), jnp.float32)),
    )(degp, xw)


def _tc_mid(e, hp, dinv, b, W):
    N, H = hp.shape
    HO = W.shape[1]
    nb = N // _RBLK

    def body(ea_ref, eb_ref, hp_ref, dinv_ref, b_ref, w_ref, out_ref):
        esum = ea_ref[:, 0:H] + eb_ref[:, 0:H] + hp_ref[...]
        h = jnp.maximum(esum * dinv_ref[...] + b_ref[...], 0.0)
        out_ref[...] = jnp.dot(h, w_ref[...],
                               preferred_element_type=jnp.float32) * dinv_ref[...]

    return pl.pallas_call(
        body,
        grid=(nb,),
        in_specs=[pl.BlockSpec((_RBLK, 128), lambda i: (i, 0)),
                  pl.BlockSpec((_RBLK, 128), lambda i: (i + nb, 0)),
                  pl.BlockSpec((_RBLK, H), lambda i: (i, 0)),
                  pl.BlockSpec((_RBLK, 1), lambda i: (i, 0)),
                  pl.BlockSpec((1, H), lambda i: (0, 0)),
                  pl.BlockSpec(W.shape, lambda i: (0, 0))],
        out_specs=pl.BlockSpec((_RBLK, HO), lambda i: (i, 0)),
        out_shape=jax.ShapeDtypeStruct((N, HO), jnp.float32),
    )(e, e, hp, dinv, b, W)


def _tc_final(e, hp, dinv, b2, W3, wsump, b3, Wc1, bc1, Wc2, bc2):
    N, H = hp.shape
    nb = N // _RBLK

    def body(ea_ref, eb_ref, hp_ref, dinv_ref, b2_ref, w3_ref, w_ref, b3_ref,
             wc1_ref, bc1_ref, wc2_ref, bc2_ref, out_ref, u_acc):
        i = pl.program_id(0)
        dinv = dinv_ref[...]
        esum = ea_ref[:, 0:H] + eb_ref[:, 0:H] + hp_ref[...]
        h2 = jnp.maximum(esum * dinv + b2_ref[...], 0.0)
        v = (w_ref[0, :, 0:1] + w_ref[1, :, 0:1] + dinv) * dinv
        u = jnp.sum(h2 * v, axis=0, keepdims=True)

        @pl.when(i == 0)
        def _():
            u_acc[...] = jnp.zeros_like(u_acc)

        u_acc[...] += u

        @pl.when(i == nb - 1)
        def _():
            g = jnp.dot(u_acc[...] * jnp.float32(1.0 / N), w3_ref[...],
                        preferred_element_type=jnp.float32) + b3_ref[...]
            z = jnp.maximum(jnp.dot(g, wc1_ref[...],
                                    preferred_element_type=jnp.float32)
                            + bc1_ref[...], 0.0)
            out_ref[...] = jnp.dot(z, wc2_ref[...],
                                   preferred_element_type=jnp.float32) + bc2_ref[...]

    return pl.pallas_call(
        body,
        grid=(nb,),
        in_specs=[pl.BlockSpec((_RBLK, 128), lambda i: (i, 0)),
                  pl.BlockSpec((_RBLK, 128), lambda i: (i + nb, 0)),
                  pl.BlockSpec((_RBLK, H), lambda i: (i, 0)),
                  pl.BlockSpec((_RBLK, 1), lambda i: (i, 0)),
                  pl.BlockSpec((1, H), lambda i: (0, 0)),
                  pl.BlockSpec(W3.shape, lambda i: (0, 0)),
                  pl.BlockSpec((2, _RBLK, 16), lambda i: (0, i, 0)),
                  pl.BlockSpec((1, H), lambda i: (0, 0)),
                  pl.BlockSpec(Wc1.shape, lambda i: (0, 0)),
                  pl.BlockSpec((1, Wc1.shape[1]), lambda i: (0, 0)),
                  pl.BlockSpec(Wc2.shape, lambda i: (0, 0)),
                  pl.BlockSpec((1, Wc2.shape[1]), lambda i: (0, 0))],
        out_specs=pl.BlockSpec((1, Wc2.shape[1]), lambda i: (0, 0)),
        out_shape=jax.ShapeDtypeStruct((1, Wc2.shape[1]), jnp.float32),
        scratch_shapes=[pltpu.VMEM((1, H), jnp.float32)],
    )(e, e, hp, dinv, b2, W3, wsump, b3, Wc1, bc1, Wc2, bc2)


def kernel(x, edge_index, W1, b1, W2, b2, W3, b3, Wc1, bc1, Wc2, bc2):
    N = x.shape[0]
    H = W1.shape[1]
    eidx = edge_index

    xw1 = _tc_matmul(x, W1)
    degp = _sc_degree(eidx, N)
    h1p, dinv, dinv16 = _tc_first(degp.reshape(2, N, 16), xw1)

    e1, wsump = _sc_edge_accumulate_fused(h1p, dinv16, eidx)
    h2p = _tc_mid(e1, h1p, dinv, b1.reshape(1, -1), W2)

    e2 = _sc_edge_accumulate(h2p, eidx)
    out = _tc_final(e2, h2p, dinv, b2.reshape(1, -1), W3,
                    wsump.reshape(2, N, 16), b3.reshape(1, -1), Wc1,
                    bc1.reshape(1, -1), Wc2, bc2.reshape(1, -1))
    return out


# speedup vs baseline: 1.1439x; 1.0169x over previous
import functools

import jax
import jax.numpy as jnp
from jax import lax
from jax.experimental import pallas as pl
from jax.experimental.pallas import tpu as pltpu
from jax.experimental.pallas import tpu_sc as plsc

_NC = 2
_NS = 16
_CHUNK = 128
_RBLK = 1000


def _sc_edge_accumulate(hp, eidx):
    N, H = hp.shape
    E = eidx.shape[1]
    nw = _NC * _NS
    per_w = E // nw
    rw, tail = divmod(per_w, _CHUNK)
    nbuf = 6
    n_grp = rw // nbuf
    n_rem = rw - n_grp * nbuf
    rps = N // _NS

    mesh = plsc.VectorSubcoreMesh(core_axis_name="c", subcore_axis_name="s")

    @functools.partial(
        pl.kernel,
        mesh=mesh,
        out_type=jax.ShapeDtypeStruct((2 * N, 128), jnp.float32),
        compiler_params=pltpu.CompilerParams(use_tc_tiling_on_sc=False),
        scratch_types=[
            pltpu.VMEM((per_w,), jnp.int32),
            pltpu.VMEM((per_w,), jnp.int32),
            [pltpu.VMEM((_CHUNK, H), jnp.float32)] * nbuf,
            pltpu.VMEM_SHARED((N, H), jnp.float32),
            pltpu.SemaphoreType.DMA,
        ],
    )
    def ker(hp_hbm, eidx_hbm, out_hbm,
            sidx, didx, rows, acc, sem):
        c = lax.axis_index("c")
        s = lax.axis_index("s")
        w = c * _NS + s
        zero16 = jnp.zeros((16,), jnp.float32)

        def zrow(i, carry):
            for j in range(H // 16):
                rows[0][i, pl.ds(j * 16, 16)] = zero16
            return carry

        lax.fori_loop(0, _CHUNK, zrow, 0)
        zfull, zrem = divmod(rps, _CHUNK)
        for z in range(zfull):
            pltpu.sync_copy(rows[0], acc.at[pl.ds(s * rps + z * _CHUNK, _CHUNK)])
        if zrem:
            pltpu.sync_copy(rows[0].at[pl.ds(0, zrem)],
                            acc.at[pl.ds(s * rps + zfull * _CHUNK, zrem)])
        pltpu.sync_copy(eidx_hbm.at[0, pl.ds(w * per_w, per_w)], sidx)
        pltpu.sync_copy(eidx_hbm.at[1, pl.ds(w * per_w, per_w)], didx)
        plsc.subcore_barrier()

        def run_chunks(jb, nb, size):
            handles = [
                pltpu.async_copy(
                    hp_hbm.at[sidx.at[pl.ds((jb + b) * _CHUNK, size)]],
                    rows[b].at[pl.ds(0, size)], sem)
                for b in range(nb)
            ]
            for b in range(nb):
                handles[b].wait()
                pltpu.sync_copy(
                    rows[b].at[pl.ds(0, size)],
                    acc.at[didx.at[pl.ds((jb + b) * _CHUNK, size)]],
                    add=True)

        def group(g, carry):
            run_chunks(g * nbuf, nbuf, _CHUNK)
            return carry

        lax.fori_loop(0, n_grp, group, 0)
        if n_rem:
            run_chunks(n_grp * nbuf, n_rem, _CHUNK)
        if tail:
            run_chunks(rw, 1, tail)
        plsc.subcore_barrier()

        pltpu.sync_copy(acc.at[pl.ds(s * rps, rps)],
                        out_hbm.at[pl.ds(c * N + s * rps, rps), pl.ds(0, H)])

    return ker(hp, eidx)


def _sc_edge_accumulate_fused(hp, dinv16, eidx):
    N, H = hp.shape
    W = dinv16.shape[1]
    E = eidx.shape[1]
    nw = _NC * _NS
    per_w = E // nw
    rw, tail = divmod(per_w, _CHUNK)
    nbuf = 5
    n_grp = rw // nbuf
    n_rem = rw - n_grp * nbuf
    rps = N // _NS

    mesh = plsc.VectorSubcoreMesh(core_axis_name="c", subcore_axis_name="s")

    @functools.partial(
        pl.kernel,
        mesh=mesh,
        out_type=(jax.ShapeDtypeStruct((2 * N, 128), jnp.float32),
                  jax.ShapeDtypeStruct((2 * N, W), jnp.float32)),
        compiler_params=pltpu.CompilerParams(use_tc_tiling_on_sc=False),
        scratch_types=[
            pltpu.VMEM((per_w,), jnp.int32),
            pltpu.VMEM((per_w,), jnp.int32),
            [pltpu.VMEM((_CHUNK, H), jnp.float32)] * nbuf,
            [pltpu.VMEM((_CHUNK, W), jnp.float32)] * nbuf,
            pltpu.VMEM_SHARED((N, H), jnp.float32),
            pltpu.VMEM_SHARED((N, W), jnp.float32),
            pltpu.SemaphoreType.DMA,
            pltpu.SemaphoreType.DMA,
        ],
    )
    def ker(hp_hbm, dinv_hbm, eidx_hbm, oute_hbm, outw_hbm,
            sidx, didx, rows, wrows, acce, accw, sem, semw):
        c = lax.axis_index("c")
        s = lax.axis_index("s")
        w = c * _NS + s
        zero16 = jnp.zeros((16,), jnp.float32)

        def zrow(i, carry):
            for j in range(H // 16):
                rows[0][i, pl.ds(j * 16, 16)] = zero16
            wrows[0][i, :] = zero16
            return carry

        lax.fori_loop(0, _CHUNK, zrow, 0)
        zfull, zrem = divmod(rps, _CHUNK)
        for z in range(zfull):
            pltpu.sync_copy(rows[0], acce.at[pl.ds(s * rps + z * _CHUNK, _CHUNK)])
            pltpu.sync_copy(wrows[0], accw.at[pl.ds(s * rps + z * _CHUNK, _CHUNK)])
        if zrem:
            pltpu.sync_copy(rows[0].at[pl.ds(0, zrem)],
                            acce.at[pl.ds(s * rps + zfull * _CHUNK, zrem)])
            pltpu.sync_copy(wrows[0].at[pl.ds(0, zrem)],
                            accw.at[pl.ds(s * rps + zfull * _CHUNK, zrem)])
        pltpu.sync_copy(eidx_hbm.at[0, pl.ds(w * per_w, per_w)], sidx)
        pltpu.sync_copy(eidx_hbm.at[1, pl.ds(w * per_w, per_w)], didx)
        plsc.subcore_barrier()

        def run_chunks(jb, nb, size):
            def sl(idx_ref, b):
                return idx_ref.at[pl.ds((jb + b) * _CHUNK, size)]

            eh = [pltpu.async_copy(hp_hbm.at[sl(sidx, b)],
                                   rows[b].at[pl.ds(0, size)], sem)
                  for b in range(nb)]
            wh = [pltpu.async_copy(dinv_hbm.at[sl(didx, b)],
                                   wrows[b].at[pl.ds(0, size)], semw)
                  for b in range(nb)]
            for b in range(nb):
                eh[b].wait()
                pltpu.sync_copy(rows[b].at[pl.ds(0, size)],
                                acce.at[sl(didx, b)], add=True)
                wh[b].wait()
                pltpu.sync_copy(wrows[b].at[pl.ds(0, size)],
                                accw.at[sl(sidx, b)], add=True)

        def group(g, carry):
            run_chunks(g * nbuf, nbuf, _CHUNK)
            return carry

        lax.fori_loop(0, n_grp, group, 0)
        if n_rem:
            run_chunks(n_grp * nbuf, n_rem, _CHUNK)
        if tail:
            run_chunks(rw, 1, tail)
        plsc.subcore_barrier()

        pltpu.sync_copy(acce.at[pl.ds(s * rps, rps)],
                        oute_hbm.at[pl.ds(c * N + s * rps, rps), pl.ds(0, H)])
        pltpu.sync_copy(accw.at[pl.ds(s * rps, rps)],
                        outw_hbm.at[pl.ds(c * N + s * rps, rps)])

    return ker(hp, dinv16, eidx)


def _sc_degree(eidx, n_nodes):
    N = n_nodes
    W = 16
    E = eidx.shape[1]
    nw = _NC * _NS
    per_w = E // nw
    rw, tail = divmod(per_w, _CHUNK)
    nbuf = 6
    n_grp = rw // nbuf
    n_rem = rw - n_grp * nbuf
    rps = N // _NS

    mesh = plsc.VectorSubcoreMesh(core_axis_name="c", subcore_axis_name="s")

    @functools.partial(
        pl.kernel,
        mesh=mesh,
        out_type=jax.ShapeDtypeStruct((2 * N, 128), jnp.float32),
        compiler_params=pltpu.CompilerParams(use_tc_tiling_on_sc=False),
        scratch_types=[
            pltpu.VMEM((per_w,), jnp.int32),
            pltpu.VMEM((_CHUNK, W), jnp.float32),
            pltpu.VMEM((rps, W), jnp.float32),
            pltpu.VMEM_SHARED((N, W), jnp.float32),
            pltpu.SemaphoreType.DMA,
        ],
    )
    def ker(eidx_hbm, out_hbm, didx, ones, zbuf, acc, sem):
        c = lax.axis_index("c")
        s = lax.axis_index("s")
        w = c * _NS + s
        onehot = jnp.where(lax.iota(jnp.int32, 16) == 0,
                           jnp.float32(1), jnp.float32(0))
        zero16 = jnp.zeros((16,), jnp.float32)

        def fill(i, carry):
            ones[i, :] = onehot
            return carry

        lax.fori_loop(0, _CHUNK, fill, 0)

        def zrow(i, carry):
            zbuf[i, :] = zero16
            return carry

        lax.fori_loop(0, rps, zrow, 0)
        pltpu.sync_copy(zbuf, acc.at[pl.ds(s * rps, rps)])
        pltpu.sync_copy(eidx_hbm.at[1, pl.ds(w * per_w, per_w)], didx)
        plsc.subcore_barrier()

        def run_chunks(jb, nb, size):
            handles = [
                pltpu.async_copy(
                    ones.at[pl.ds(0, size)],
                    acc.at[didx.at[pl.ds((jb + b) * _CHUNK, size)]],
                    sem, add=True)
                for b in range(nb)
            ]
            for h in handles:
                h.wait()

        def body(g, carry):
            run_chunks(g * nbuf, nbuf, _CHUNK)
            return carry

        lax.fori_loop(0, n_grp, body, 0)
        if n_rem:
            run_chunks(n_grp * nbuf, n_rem, _CHUNK)
        if tail:
            run_chunks(rw, 1, tail)
        plsc.subcore_barrier()

        pltpu.sync_copy(acc.at[pl.ds(s * rps, rps)],
                        out_hbm.at[pl.ds(c * N + s * rps, rps), pl.ds(0, W)])

    return ker(eidx)


def _tc_matmul(x, W1):
    N = x.shape[0]
    H = W1.shape[1]

    def body(x_ref, w_ref, out_ref):
        out_ref[...] = jnp.dot(x_ref[...], w_ref[...],
                               preferred_element_type=jnp.float32)

    nb = N // _RBLK
    return pl.pallas_call(
        body,
        grid=(nb,),
        in_specs=[pl.BlockSpec((_RBLK, x.shape[1]), lambda i: (i, 0)),
                  pl.BlockSpec(W1.shape, lambda i: (0, 0))],
        out_specs=pl.BlockSpec((_RBLK, H), lambda i: (i, 0)),
        out_shape=jax.ShapeDtypeStruct((N, H), jnp.float32),
    )(x, W1)


def _tc_first(degp, xw):
    N, H = xw.shape
    nb = N // _RBLK

    def body(da_ref, db_ref, xw_ref, hp_ref, dinv_ref, dinv16_ref):
        deg = da_ref[:, 0:1] + db_ref[:, 0:1] + 1.0
        dinv = lax.rsqrt(deg)
        dinv_ref[...] = dinv
        dinv16_ref[...] = jnp.broadcast_to(dinv, (_RBLK, 16))
        hp_ref[...] = xw_ref[...] * dinv

    return pl.pallas_call(
        body,
        grid=(nb,),
        in_specs=[pl.BlockSpec((_RBLK, 128), lambda i: (i, 0)),
                  pl.BlockSpec((_RBLK, 128), lambda i: (i + nb, 0)),
                  pl.BlockSpec((_RBLK, H), lambda i: (i, 0))],
        out_specs=(pl.BlockSpec((_RBLK, H), lambda i: (i, 0)),
                   pl.BlockSpec((_RBLK, 1), lambda i: (i, 0)),
                   pl.BlockSpec((_RBLK, 16), lambda i: (i, 0))),
        out_shape=(jax.ShapeDtypeStruct((N, H), jnp.float32),
                   jax.ShapeDtypeStruct((N, 1), jnp.float32),
                   jax.ShapeDtypeStruct((N, 16), jnp.float32)),
    )(degp, degp, xw)


def _tc_mid(e, hp, dinv, b, W):
    N, H = hp.shape
    HO = W.shape[1]
    nb = N // _RBLK

    def body(ea_ref, eb_ref, hp_ref, dinv_ref, b_ref, w_ref, out_ref):
        esum = ea_ref[:, 0:H] + eb_ref[:, 0:H] + hp_ref[...]
        h = jnp.maximum(esum * dinv_ref[...] + b_ref[...], 0.0)
        out_ref[...] = jnp.dot(h, w_ref[...],
                               preferred_element_type=jnp.float32) * dinv_ref[...]

    return pl.pallas_call(
        body,
        grid=(nb,),
        in_specs=[pl.BlockSpec((_RBLK, 128), lambda i: (i, 0)),
                  pl.BlockSpec((_RBLK, 128), lambda i: (i + nb, 0)),
                  pl.BlockSpec((_RBLK, H), lambda i: (i, 0)),
                  pl.BlockSpec((_RBLK, 1), lambda i: (i, 0)),
                  pl.BlockSpec((1, H), lambda i: (0, 0)),
                  pl.BlockSpec(W.shape, lambda i: (0, 0))],
        out_specs=pl.BlockSpec((_RBLK, HO), lambda i: (i, 0)),
        out_shape=jax.ShapeDtypeStruct((N, HO), jnp.float32),
    )(e, e, hp, dinv, b, W)


def _tc_final(e, hp, dinv, b2, W3, wsump, b3, Wc1, bc1, Wc2, bc2):
    N, H = hp.shape
    nb = N // _RBLK

    def body(ea_ref, eb_ref, hp_ref, dinv_ref, b2_ref, w3_ref, w_ref, b3_ref,
             wc1_ref, bc1_ref, wc2_ref, bc2_ref, out_ref, u_acc):
        i = pl.program_id(0)
        dinv = dinv_ref[...]
        esum = ea_ref[:, 0:H] + eb_ref[:, 0:H] + hp_ref[...]
        h2 = jnp.maximum(esum * dinv + b2_ref[...], 0.0)
        v = (w_ref[0, :, 0:1] + w_ref[1, :, 0:1] + dinv) * dinv
        u = jnp.sum(h2 * v, axis=0, keepdims=True)

        @pl.when(i == 0)
        def _():
            u_acc[...] = jnp.zeros_like(u_acc)

        u_acc[...] += u

        @pl.when(i == nb - 1)
        def _():
            g = jnp.dot(u_acc[...] * jnp.float32(1.0 / N), w3_ref[...],
                        preferred_element_type=jnp.float32) + b3_ref[...]
            z = jnp.maximum(jnp.dot(g, wc1_ref[...],
                                    preferred_element_type=jnp.float32)
                            + bc1_ref[...], 0.0)
            out_ref[...] = jnp.dot(z, wc2_ref[...],
                                   preferred_element_type=jnp.float32) + bc2_ref[...]

    return pl.pallas_call(
        body,
        grid=(nb,),
        in_specs=[pl.BlockSpec((_RBLK, 128), lambda i: (i, 0)),
                  pl.BlockSpec((_RBLK, 128), lambda i: (i + nb, 0)),
                  pl.BlockSpec((_RBLK, H), lambda i: (i, 0)),
                  pl.BlockSpec((_RBLK, 1), lambda i: (i, 0)),
                  pl.BlockSpec((1, H), lambda i: (0, 0)),
                  pl.BlockSpec(W3.shape, lambda i: (0, 0)),
                  pl.BlockSpec((2, _RBLK, 16), lambda i: (0, i, 0)),
                  pl.BlockSpec((1, H), lambda i: (0, 0)),
                  pl.BlockSpec(Wc1.shape, lambda i: (0, 0)),
                  pl.BlockSpec((1, Wc1.shape[1]), lambda i: (0, 0)),
                  pl.BlockSpec(Wc2.shape, lambda i: (0, 0)),
                  pl.BlockSpec((1, Wc2.shape[1]), lambda i: (0, 0))],
        out_specs=pl.BlockSpec((1, Wc2.shape[1]), lambda i: (0, 0)),
        out_shape=jax.ShapeDtypeStruct((1, Wc2.shape[1]), jnp.float32),
        scratch_shapes=[pltpu.VMEM((1, H), jnp.float32)],
    )(e, e, hp, dinv, b2, W3, wsump, b3, Wc1, bc1, Wc2, bc2)


def kernel(x, edge_index, W1, b1, W2, b2, W3, b3, Wc1, bc1, Wc2, bc2):
    N = x.shape[0]
    H = W1.shape[1]
    eidx = edge_index

    xw1 = _tc_matmul(x, W1)
    degp = _sc_degree(eidx, N)
    h1p, dinv, dinv16 = _tc_first(degp, xw1)

    e1, wsump = _sc_edge_accumulate_fused(h1p, dinv16, eidx)
    h2p = _tc_mid(e1, h1p, dinv, b1.reshape(1, -1), W2)

    e2 = _sc_edge_accumulate(h2p, eidx)
    out = _tc_final(e2, h2p, dinv, b2.reshape(1, -1), W3,
                    wsump.reshape(2, N, 16), b3.reshape(1, -1), Wc1,
                    bc1.reshape(1, -1), Wc2, bc2.reshape(1, -1))
    return out
